# baseline (speedup 1.0000x reference)
import functools

import jax
import jax.numpy as jnp
from jax import lax
from jax.experimental import pallas as pl
from jax.experimental.pallas import tpu as pltpu
from jax.experimental.pallas import tpu_sc as plsc

_ROW_BLK = 1024
_TILE_R = 128


def _sc_emb_fill(temb_hbm, idx_hbm, out_hbm, idx_v, tile_v, gsem, ssem):
    info = plsc.get_sparse_core_info()
    nc, ns = info.num_cores, info.num_subcores
    nw = nc * ns
    n, d_out = out_hbm.shape
    d_emb = temb_hbm.shape[-1]
    d_in = d_out - d_emb
    rows_per_w = n // nw
    wid = lax.axis_index("s") * nc + lax.axis_index("c")
    base = wid * rows_per_w

    pltpu.sync_copy(idx_hbm, idx_v)
    pltpu.async_copy(temb_hbm.at[idx_v], tile_v, gsem).wait()

    copies = [
        pltpu.async_copy(
            tile_v,
            out_hbm.at[pl.ds(base + t * _TILE_R, _TILE_R), pl.ds(d_in, d_emb)],
            ssem,
        )
        for t in range(rows_per_w // _TILE_R)
    ]
    for cp in copies:
        cp.start()
    for cp in copies:
        cp.wait()


def _tc_fill_x(x_ref, out0_ref, out_ref):
    del out0_ref
    out_ref[...] = x_ref[...]


def kernel(x, type_emb, index):
    b, s, d = x.shape
    n = b * s
    d_emb = type_emb.shape[-1]
    x2 = x.reshape(n, d)
    idx_arr = jnp.full((_TILE_R,), index, dtype=jnp.int32)

    mesh = plsc.VectorSubcoreMesh(core_axis_name="c", subcore_axis_name="s")
    sc_fill = pl.kernel(
        _sc_emb_fill,
        out_type=jax.ShapeDtypeStruct((n, d + d_emb), x.dtype),
        mesh=mesh,
        scratch_types=[
            pltpu.VMEM((_TILE_R,), jnp.int32),
            pltpu.VMEM((_TILE_R, d_emb), x.dtype),
            pltpu.SemaphoreType.DMA,
            pltpu.SemaphoreType.DMA,
        ],
    )
    out0 = sc_fill(type_emb, idx_arr)

    out = pl.pallas_call(
        _tc_fill_x,
        grid=(n // _ROW_BLK,),
        in_specs=[
            pl.BlockSpec((_ROW_BLK, d), lambda i: (i, 0)),
            pl.BlockSpec(memory_space=pl.ANY),
        ],
        out_specs=pl.BlockSpec((_ROW_BLK, d), lambda i: (i, 0)),
        out_shape=jax.ShapeDtypeStruct((n, d + d_emb), x.dtype),
        input_output_aliases={1: 0},
    )(x2, out0)
    return out.reshape(b, s, d + d_emb)


# speedup vs baseline: 2.2153x; 2.2153x over previous
import jax
import jax.numpy as jnp
from jax.experimental import pallas as pl
from jax.experimental.pallas import tpu as pltpu

_ROW_BLK = 1024


def _cat_kernel(idx_ref, x_ref, temb_ref, out_ref):
    d_in = x_ref.shape[-1]
    d_emb = temb_ref.shape[-1]
    out_ref[:, :d_in] = x_ref[...]
    idx = idx_ref[0]
    row = temb_ref[pl.ds(idx, 1), :]
    out_ref[:, d_in:] = jnp.broadcast_to(row, (out_ref.shape[0], d_emb))


def kernel(x, type_emb, index):
    b, s, d = x.shape
    n = b * s
    d_emb = type_emb.shape[-1]
    x2 = x.reshape(n, d)
    idx = jnp.asarray(index, jnp.int32).reshape((1,))
    out = pl.pallas_call(
        _cat_kernel,
        grid_spec=pltpu.PrefetchScalarGridSpec(
            num_scalar_prefetch=1,
            grid=(n // _ROW_BLK,),
            in_specs=[
                pl.BlockSpec((_ROW_BLK, d), lambda i, s_ref: (i, 0)),
                pl.BlockSpec(type_emb.shape, lambda i, s_ref: (0, 0)),
            ],
            out_specs=pl.BlockSpec((_ROW_BLK, d + d_emb), lambda i, s_ref: (i, 0)),
        ),
        out_shape=jax.ShapeDtypeStruct((n, d + d_emb), x.dtype),
    )(idx, x2, type_emb)
    return out.reshape(b, s, d + d_emb)
